# baseline (speedup 1.0000x reference)
import jax
import jax.numpy as jnp
from jax import lax
from jax.experimental import pallas as pl
from jax.experimental.pallas import tpu as pltpu
from jax.experimental.pallas import tpu_sc as plsc

EMBED = 64
PADDED_W = 128
BATCH = 16384
NC = 2
NS = 16
NW = NC * NS
B_PER_W = BATCH // NW
IDX_CHUNK = 128
N_CHUNKS = B_PER_W // IDX_CHUNK
GROUPS_PER_CHUNK = IDX_CHUNK // 16

PAD_BLK = 4000
N_PAD_BLKS = 100000 // PAD_BLK


def _tc_pad_body(t_ref, o_ref):
    o_ref[:, :65] = t_ref[...]


def _tc_pad(table):
    n_rows = table.shape[0]
    return pl.pallas_call(
        _tc_pad_body,
        grid=(N_PAD_BLKS,),
        in_specs=[pl.BlockSpec((PAD_BLK, 65), lambda i: (i, 0))],
        out_specs=pl.BlockSpec((PAD_BLK, PADDED_W), lambda i: (i, 0)),
        out_shape=jax.ShapeDtypeStruct((n_rows, PADDED_W), jnp.float32),
    )(table)


def _sc_body(att_hbm, table_hbm, ent_hbm, out_hbm,
             idx_v, rows_v, ent_v, pad_v, out_v,
             sg0, sg1, sent):
    wid = lax.axis_index("s") * NC + lax.axis_index("c")
    base = wid * B_PER_W
    gsems = [sg0, sg1]

    pltpu.sync_copy(att_hbm.at[wid], idx_v)

    def start_gather(j):
        return pltpu.async_copy(
            table_hbm.at[idx_v.at[j]], rows_v.at[j % 2], gsems[j % 2])

    gathers = [start_gather(0), start_gather(1)]
    ecopy = pltpu.async_copy(ent_hbm.at[pl.ds(base, B_PER_W)], ent_v, sent)
    ecopy.wait()

    lanes = lax.iota(jnp.int32, 16)
    one = jnp.full((16,), 1, jnp.int32)

    for j in range(N_CHUNKS):
        gathers[j].wait()
        buf = rows_v.at[j % 2]

        def group_body(g, _, j=j, buf=buf):
            row0 = g * 16
            for r in range(16):
                row = row0 + r
                erow = j * IDX_CHUNK + row
                acc = buf[row, pl.ds(0, 16)] * ent_v[erow, pl.ds(0, 16)]
                for q in range(1, 4):
                    acc = acc + (buf[row, pl.ds(16 * q, 16)]
                                 * ent_v[erow, pl.ds(16 * q, 16)])
                pad_v[r, pl.ds(0, 16)] = acc
            tot = plsc.load_gather(pad_v, [lanes, jnp.full((16,), 0, jnp.int32)])
            for c in range(1, 16):
                tot = tot + plsc.load_gather(
                    pad_v, [lanes, jnp.full((16,), c, jnp.int32)])
            bias = plsc.load_gather(buf, [row0 + lanes, one])
            logits = tot + bias
            sig = 1.0 / (1.0 + jnp.exp(-logits))
            out_v[pl.ds(j * IDX_CHUNK + row0, 16)] = sig
            return 0

        lax.fori_loop(0, GROUPS_PER_CHUNK, group_body, 0)
        if j + 2 < N_CHUNKS:
            gathers.append(start_gather(j + 2))

    pltpu.sync_copy(out_v, out_hbm.at[pl.ds(base, B_PER_W)])


def _sc_fused(att, table_pad, ent):
    mesh = plsc.VectorSubcoreMesh(core_axis_name="c", subcore_axis_name="s")
    kern = pl.kernel(
        _sc_body,
        mesh=mesh,
        out_type=jax.ShapeDtypeStruct((BATCH,), jnp.float32),
        scratch_types=[
            pltpu.VMEM((N_CHUNKS, IDX_CHUNK), jnp.int32),
            pltpu.VMEM((2, IDX_CHUNK, PADDED_W), jnp.float32),
            pltpu.VMEM((B_PER_W, EMBED), jnp.float32),
            pltpu.VMEM((16, 17), jnp.float32),
            pltpu.VMEM((B_PER_W,), jnp.float32),
            pltpu.SemaphoreType.DMA,
            pltpu.SemaphoreType.DMA,
            pltpu.SemaphoreType.DMA,
        ],
        compiler_params=pltpu.CompilerParams(needs_layout_passes=False),
    )
    return kern(att.reshape(NW, N_CHUNKS, IDX_CHUNK), table_pad, ent)


def kernel(ent, att, att_embed_weight):
    att = att.astype(jnp.int32)
    table_pad = _tc_pad(att_embed_weight)
    return _sc_fused(att, table_pad, ent)


# speedup vs baseline: 1.0223x; 1.0223x over previous
import jax
import jax.numpy as jnp
from jax import lax
from jax.experimental import pallas as pl
from jax.experimental.pallas import tpu as pltpu
from jax.experimental.pallas import tpu_sc as plsc

EMBED = 64
PADDED_W = 128
BATCH = 16384
NC = 2
NS = 16
NW = NC * NS
B_PER_W = BATCH // NW
IDX_CHUNK = 128
N_CHUNKS = B_PER_W // IDX_CHUNK
GROUPS_PER_CHUNK = IDX_CHUNK // 16

PAD_BLK = 10000
N_PAD_BLKS = 100000 // PAD_BLK


def _tc_pad_body(t_ref, o_ref):
    o_ref[:, :65] = t_ref[...]


def _tc_pad(table):
    n_rows = table.shape[0]
    return pl.pallas_call(
        _tc_pad_body,
        grid=(N_PAD_BLKS,),
        in_specs=[pl.BlockSpec((PAD_BLK, 65), lambda i: (i, 0))],
        out_specs=pl.BlockSpec((PAD_BLK, PADDED_W), lambda i: (i, 0)),
        out_shape=jax.ShapeDtypeStruct((n_rows, PADDED_W), jnp.float32),
    )(table)


def _sc_body(att_hbm, table_hbm, ent_hbm, out_hbm,
             idx_v, rows_v, ent_v, pad_v, out_v,
             sg0, sg1, sent):
    wid = lax.axis_index("s") * NC + lax.axis_index("c")
    base = wid * B_PER_W
    gsems = [sg0, sg1]

    pltpu.sync_copy(att_hbm.at[wid], idx_v)

    def start_gather(j):
        return pltpu.async_copy(
            table_hbm.at[idx_v.at[j]], rows_v.at[j % 2], gsems[j % 2])

    gathers = [start_gather(0), start_gather(1)]
    ecopy = pltpu.async_copy(ent_hbm.at[pl.ds(base, B_PER_W)], ent_v, sent)
    ecopy.wait()

    lanes = lax.iota(jnp.int32, 16)
    one = jnp.full((16,), 1, jnp.int32)

    for j in range(N_CHUNKS):
        gathers[j].wait()
        buf = rows_v.at[j % 2]

        def group_body(g, _, j=j, buf=buf):
            row0 = g * 16
            for r in range(16):
                row = row0 + r
                erow = j * IDX_CHUNK + row
                acc = buf[row, pl.ds(0, 16)] * ent_v[erow, pl.ds(0, 16)]
                for q in range(1, 4):
                    acc = acc + (buf[row, pl.ds(16 * q, 16)]
                                 * ent_v[erow, pl.ds(16 * q, 16)])
                pad_v[r, pl.ds(0, 16)] = acc
            tot = plsc.load_gather(pad_v, [lanes, jnp.full((16,), 0, jnp.int32)])
            for c in range(1, 16):
                tot = tot + plsc.load_gather(
                    pad_v, [lanes, jnp.full((16,), c, jnp.int32)])
            bias = plsc.load_gather(buf, [row0 + lanes, one])
            logits = tot + bias
            sig = 1.0 / (1.0 + jnp.exp(-logits))
            out_v[pl.ds(j * IDX_CHUNK + row0, 16)] = sig
            return 0

        lax.fori_loop(0, GROUPS_PER_CHUNK, group_body, 0)
        if j + 2 < N_CHUNKS:
            gathers.append(start_gather(j + 2))

    pltpu.sync_copy(out_v, out_hbm.at[pl.ds(base, B_PER_W)])


def _sc_fused(att, table_pad, ent):
    mesh = plsc.VectorSubcoreMesh(core_axis_name="c", subcore_axis_name="s")
    kern = pl.kernel(
        _sc_body,
        mesh=mesh,
        out_type=jax.ShapeDtypeStruct((BATCH,), jnp.float32),
        scratch_types=[
            pltpu.VMEM((N_CHUNKS, IDX_CHUNK), jnp.int32),
            pltpu.VMEM((2, IDX_CHUNK, PADDED_W), jnp.float32),
            pltpu.VMEM((B_PER_W, EMBED), jnp.float32),
            pltpu.VMEM((16, 17), jnp.float32),
            pltpu.VMEM((B_PER_W,), jnp.float32),
            pltpu.SemaphoreType.DMA,
            pltpu.SemaphoreType.DMA,
            pltpu.SemaphoreType.DMA,
        ],
        compiler_params=pltpu.CompilerParams(needs_layout_passes=False),
    )
    return kern(att.reshape(NW, N_CHUNKS, IDX_CHUNK), table_pad, ent)


def kernel(ent, att, att_embed_weight):
    att = att.astype(jnp.int32)
    table_pad = _tc_pad(att_embed_weight)
    return _sc_fused(att, table_pad, ent)
